# Initial kernel scaffold; baseline (speedup 1.0000x reference)
import functools

import jax
import jax.numpy as jnp
from jax import lax
from jax.experimental import pallas as pl
from jax.experimental.pallas import tpu as pltpu
from jax.experimental.pallas import tpu_sc as plsc

N_NODES = 10000
N_EDGES = 320000
D = 128

NUM_CORES = 2
NUM_SUBCORES = 16
NUM_WORKERS = NUM_CORES * NUM_SUBCORES
EDGES_PER_WORKER = N_EDGES // NUM_WORKERS
CHUNK = 80
NUM_CHUNKS = EDGES_PER_WORKER // CHUNK
ROWS_PER_SUBCORE = N_NODES // NUM_SUBCORES

ROW_BLOCK = 1000
N_ROW_BLOCKS = N_NODES // ROW_BLOCK


def _mm_relu_body(x_ref, w_ref, o_ref):
    o_ref[...] = jnp.maximum(
        jnp.dot(x_ref[...], w_ref[...], preferred_element_type=jnp.float32), 0.0
    )


def _mm_sum_body(a_ref, b_ref, w_ref, o_ref):
    o_ref[...] = jnp.dot(
        a_ref[...] + b_ref[...], w_ref[...], preferred_element_type=jnp.float32
    )


def _edge_accumulate(h, src3, dst3, zeros):
    mesh = plsc.VectorSubcoreMesh(core_axis_name="c", subcore_axis_name="s")

    @functools.partial(
        pl.kernel,
        mesh=mesh,
        out_type=jax.ShapeDtypeStruct((NUM_CORES, N_NODES, D), jnp.float32),
        scratch_types=[
            pltpu.VMEM((NUM_CHUNKS, CHUNK), jnp.int32),
            pltpu.VMEM((NUM_CHUNKS, CHUNK), jnp.int32),
            pltpu.VMEM((CHUNK, D), jnp.float32),
            pltpu.VMEM_SHARED((N_NODES, D), jnp.float32),
            pltpu.SemaphoreType.DMA,
        ],
    )
    def k(h_hbm, src_hbm, dst_hbm, zeros_hbm, out_hbm,
          src_v, dst_v, rows_v, acc, sem):
        c = lax.axis_index("c")
        s = lax.axis_index("s")
        w = c * NUM_SUBCORES + s
        pltpu.sync_copy(
            zeros_hbm.at[pl.ds(s * ROWS_PER_SUBCORE, ROWS_PER_SUBCORE)],
            acc.at[pl.ds(s * ROWS_PER_SUBCORE, ROWS_PER_SUBCORE)],
        )
        pltpu.sync_copy(src_hbm.at[w], src_v)
        pltpu.sync_copy(dst_hbm.at[w], dst_v)
        plsc.subcore_barrier()

        def step(j, carry):
            pltpu.async_copy(h_hbm.at[src_v.at[j]], rows_v, sem).wait()
            pltpu.sync_copy(rows_v, acc.at[dst_v.at[j]], add=True)
            return carry

        lax.fori_loop(0, NUM_CHUNKS, step, 0, unroll=False)
        plsc.subcore_barrier()
        pltpu.sync_copy(
            acc.at[pl.ds(s * ROWS_PER_SUBCORE, ROWS_PER_SUBCORE)],
            out_hbm.at[c, pl.ds(s * ROWS_PER_SUBCORE, ROWS_PER_SUBCORE)],
        )

    return k(h, src3, dst3, zeros)


@jax.jit
def kernel(x, edge_label, edge_label_index, W1, W2):
    del edge_label
    h = pl.pallas_call(
        _mm_relu_body,
        grid=(N_ROW_BLOCKS,),
        in_specs=[
            pl.BlockSpec((ROW_BLOCK, D), lambda i: (i, 0)),
            pl.BlockSpec((D, D), lambda i: (0, 0)),
        ],
        out_specs=pl.BlockSpec((ROW_BLOCK, D), lambda i: (i, 0)),
        out_shape=jax.ShapeDtypeStruct((N_NODES, D), jnp.float32),
    )(x, W1)

    src3 = edge_label_index[0].reshape(NUM_WORKERS, NUM_CHUNKS, CHUNK)
    dst3 = edge_label_index[1].reshape(NUM_WORKERS, NUM_CHUNKS, CHUNK)
    zeros = jnp.zeros((N_NODES, D), jnp.float32)
    partials = _edge_accumulate(h, src3, dst3, zeros)

    s = pl.pallas_call(
        _mm_sum_body,
        grid=(N_ROW_BLOCKS,),
        in_specs=[
            pl.BlockSpec((ROW_BLOCK, D), lambda i: (i, 0)),
            pl.BlockSpec((ROW_BLOCK, D), lambda i: (i, 0)),
            pl.BlockSpec((D, D), lambda i: (0, 0)),
        ],
        out_specs=pl.BlockSpec((ROW_BLOCK, D), lambda i: (i, 0)),
        out_shape=jax.ShapeDtypeStruct((N_NODES, D), jnp.float32),
    )(partials[0], partials[1], W2)
    return s


# speedup vs baseline: 7.1640x; 7.1640x over previous
import functools

import jax
import jax.numpy as jnp
from jax import lax
from jax.experimental import pallas as pl
from jax.experimental.pallas import tpu as pltpu
from jax.experimental.pallas import tpu_sc as plsc

N_NODES = 10000
N_EDGES = 320000
D = 128

NUM_CORES = 2
NUM_SUBCORES = 16
NUM_WORKERS = NUM_CORES * NUM_SUBCORES
EDGES_PER_WORKER = N_EDGES // NUM_WORKERS
CHUNK = 80
NUM_CHUNKS = EDGES_PER_WORKER // CHUNK
N_PAD = 10240
ROWS_PER_SUBCORE = N_PAD // NUM_SUBCORES

ROW_BLOCK = 1000
N_ROW_BLOCKS = N_NODES // ROW_BLOCK


def _mm_relu_body(x_ref, w_ref, o_ref):
    o_ref[...] = jnp.maximum(
        jnp.dot(x_ref[...], w_ref[...], preferred_element_type=jnp.float32), 0.0
    )


def _mm_sum_body(a_ref, b_ref, w_ref, o_ref):
    o_ref[...] = jnp.dot(
        a_ref[...] + b_ref[...], w_ref[...], preferred_element_type=jnp.float32
    )


def _edge_accumulate(h, src3, dst3, zeros):
    mesh = plsc.VectorSubcoreMesh(core_axis_name="c", subcore_axis_name="s")

    @functools.partial(
        pl.kernel,
        mesh=mesh,
        out_type=jax.ShapeDtypeStruct((NUM_CORES, N_PAD, D), jnp.float32),
        scratch_types=[
            pltpu.VMEM((NUM_CHUNKS, CHUNK), jnp.int32),
            pltpu.VMEM((NUM_CHUNKS, CHUNK), jnp.int32),
            pltpu.VMEM((CHUNK, D), jnp.float32),
            pltpu.VMEM_SHARED((N_PAD, D), jnp.float32),
            pltpu.SemaphoreType.DMA,
        ],
    )
    def k(h_hbm, src_hbm, dst_hbm, zeros_hbm, out_hbm,
          src_v, dst_v, rows_v, acc, sem):
        c = lax.axis_index("c")
        s = lax.axis_index("s")
        w = c * NUM_SUBCORES + s
        pltpu.sync_copy(
            zeros_hbm.at[pl.ds(s * ROWS_PER_SUBCORE, ROWS_PER_SUBCORE)],
            acc.at[pl.ds(s * ROWS_PER_SUBCORE, ROWS_PER_SUBCORE)],
        )
        pltpu.sync_copy(src_hbm.at[w], src_v)
        pltpu.sync_copy(dst_hbm.at[w], dst_v)
        plsc.subcore_barrier()

        def step(j, carry):
            pltpu.async_copy(h_hbm.at[src_v.at[j]], rows_v, sem).wait()
            pltpu.sync_copy(rows_v, acc.at[dst_v.at[j]], add=True)
            return carry

        lax.fori_loop(0, NUM_CHUNKS, step, 0, unroll=False)
        plsc.subcore_barrier()
        pltpu.sync_copy(
            acc.at[pl.ds(s * ROWS_PER_SUBCORE, ROWS_PER_SUBCORE)],
            out_hbm.at[c, pl.ds(s * ROWS_PER_SUBCORE, ROWS_PER_SUBCORE)],
        )

    return k(h, src3, dst3, zeros)


@jax.jit
def kernel(x, edge_label, edge_label_index, W1, W2):
    del edge_label
    h = pl.pallas_call(
        _mm_relu_body,
        grid=(N_ROW_BLOCKS,),
        in_specs=[
            pl.BlockSpec((ROW_BLOCK, D), lambda i: (i, 0)),
            pl.BlockSpec((D, D), lambda i: (0, 0)),
        ],
        out_specs=pl.BlockSpec((ROW_BLOCK, D), lambda i: (i, 0)),
        out_shape=jax.ShapeDtypeStruct((N_NODES, D), jnp.float32),
    )(x, W1)

    src3 = edge_label_index[0].reshape(NUM_WORKERS, NUM_CHUNKS, CHUNK)
    dst3 = edge_label_index[1].reshape(NUM_WORKERS, NUM_CHUNKS, CHUNK)
    zeros = jnp.zeros((N_PAD, D), jnp.float32)
    partials = _edge_accumulate(h, src3, dst3, zeros)[:, :N_NODES, :]

    s = pl.pallas_call(
        _mm_sum_body,
        grid=(N_ROW_BLOCKS,),
        in_specs=[
            pl.BlockSpec((ROW_BLOCK, D), lambda i: (i, 0)),
            pl.BlockSpec((ROW_BLOCK, D), lambda i: (i, 0)),
            pl.BlockSpec((D, D), lambda i: (0, 0)),
        ],
        out_specs=pl.BlockSpec((ROW_BLOCK, D), lambda i: (i, 0)),
        out_shape=jax.ShapeDtypeStruct((N_NODES, D), jnp.float32),
    )(partials[0], partials[1], W2)
    return s


# speedup vs baseline: 10.6774x; 1.4904x over previous
import functools

import jax
import jax.numpy as jnp
from jax import lax
from jax.experimental import pallas as pl
from jax.experimental.pallas import tpu as pltpu
from jax.experimental.pallas import tpu_sc as plsc

N_NODES = 10000
N_EDGES = 320000
D = 128

NUM_CORES = 2
NUM_SUBCORES = 16
NUM_WORKERS = NUM_CORES * NUM_SUBCORES
CHUNK = 128
G_CHUNKS = N_EDGES // CHUNK
BASE_CHUNKS = G_CHUNKS // NUM_WORKERS
G_PAD = 2560
MAX_CHUNKS = 80
N_PAD = 10240
ROWS_PER_SUBCORE = N_PAD // NUM_SUBCORES

ROW_BLOCK = 1000
N_ROW_BLOCKS = N_NODES // ROW_BLOCK


def _mm_relu_body(x_ref, w_ref, o_ref):
    o_ref[...] = jnp.maximum(
        jnp.dot(x_ref[...], w_ref[...], preferred_element_type=jnp.float32), 0.0
    )


def _mm_sum_body(a_ref, b_ref, w_ref, o_ref):
    o_ref[...] = jnp.dot(
        a_ref[...] + b_ref[...], w_ref[...], preferred_element_type=jnp.float32
    )


def _edge_accumulate(h, src2, dst2, zeros):
    mesh = plsc.VectorSubcoreMesh(core_axis_name="c", subcore_axis_name="s")

    @functools.partial(
        pl.kernel,
        mesh=mesh,
        out_type=jax.ShapeDtypeStruct((NUM_CORES, N_PAD, D), jnp.float32),
        scratch_types=[
            pltpu.VMEM((MAX_CHUNKS, 1, CHUNK), jnp.int32),
            pltpu.VMEM((2, 1, CHUNK), jnp.int32),
            pltpu.VMEM((CHUNK, D), jnp.float32),
            pltpu.VMEM((CHUNK, D), jnp.float32),
            pltpu.VMEM_SHARED((N_PAD, D), jnp.float32),
            pltpu.SemaphoreType.DMA,
            pltpu.SemaphoreType.DMA,
            pltpu.SemaphoreType.DMA,
            pltpu.SemaphoreType.DMA,
        ],
    )
    def k(h_hbm, src_hbm, dst_hbm, zeros_hbm, out_hbm,
          src_v, dstb, rows0, rows1, acc, g0, g1, d0, d1):
        c = lax.axis_index("c")
        s = lax.axis_index("s")
        w = c * NUM_SUBCORES + s
        start = BASE_CHUNKS * w + jnp.minimum(w, 4)
        cnt = BASE_CHUNKS + jnp.where(w < 4, 1, 0)
        pltpu.sync_copy(
            zeros_hbm.at[pl.ds(s * ROWS_PER_SUBCORE, ROWS_PER_SUBCORE)],
            acc.at[pl.ds(s * ROWS_PER_SUBCORE, ROWS_PER_SUBCORE)],
        )
        pltpu.sync_copy(src_hbm.at[pl.ds(start, MAX_CHUNKS)], src_v)
        plsc.subcore_barrier()

        pltpu.async_copy(dst_hbm.at[start], dstb.at[0], d0)
        pltpu.async_copy(dst_hbm.at[start + 1], dstb.at[1], d1)
        pltpu.async_copy(h_hbm.at[src_v.at[0, 0]], rows0, g0)

        def step(j, carry):
            even = (j % 2) == 0
            more1 = j + 1 < cnt
            more2 = j + 2 < cnt

            @pl.when(even)
            def _():
                pltpu.make_async_copy(h_hbm.at[src_v.at[j, 0]], rows0, g0).wait()

            @pl.when(jnp.logical_not(even))
            def _():
                pltpu.make_async_copy(h_hbm.at[src_v.at[j, 0]], rows1, g1).wait()

            @pl.when(jnp.logical_and(even, more1))
            def _():
                pltpu.async_copy(h_hbm.at[src_v.at[j + 1, 0]], rows1, g1)

            @pl.when(jnp.logical_and(jnp.logical_not(even), more1))
            def _():
                pltpu.async_copy(h_hbm.at[src_v.at[j + 1, 0]], rows0, g0)

            @pl.when(even)
            def _():
                pltpu.make_async_copy(dst_hbm.at[start + j], dstb.at[0], d0).wait()
                pltpu.sync_copy(rows0, acc.at[dstb.at[0, 0]], add=True)

            @pl.when(jnp.logical_not(even))
            def _():
                pltpu.make_async_copy(dst_hbm.at[start + j], dstb.at[1], d1).wait()
                pltpu.sync_copy(rows1, acc.at[dstb.at[1, 0]], add=True)

            @pl.when(jnp.logical_and(even, more2))
            def _():
                pltpu.async_copy(dst_hbm.at[start + j + 2], dstb.at[0], d0)

            @pl.when(jnp.logical_and(jnp.logical_not(even), more2))
            def _():
                pltpu.async_copy(dst_hbm.at[start + j + 2], dstb.at[1], d1)

            return carry

        lax.fori_loop(0, cnt, step, 0, unroll=False)
        plsc.subcore_barrier()
        pltpu.sync_copy(
            acc.at[pl.ds(s * ROWS_PER_SUBCORE, ROWS_PER_SUBCORE)],
            out_hbm.at[c, pl.ds(s * ROWS_PER_SUBCORE, ROWS_PER_SUBCORE)],
        )

    return k(h, src2, dst2, zeros)


@jax.jit
def kernel(x, edge_label, edge_label_index, W1, W2):
    del edge_label
    h = pl.pallas_call(
        _mm_relu_body,
        grid=(N_ROW_BLOCKS,),
        in_specs=[
            pl.BlockSpec((ROW_BLOCK, D), lambda i: (i, 0)),
            pl.BlockSpec((D, D), lambda i: (0, 0)),
        ],
        out_specs=pl.BlockSpec((ROW_BLOCK, D), lambda i: (i, 0)),
        out_shape=jax.ShapeDtypeStruct((N_NODES, D), jnp.float32),
    )(x, W1)

    src2 = edge_label_index[0].reshape(G_CHUNKS, 1, CHUNK)
    src2 = jnp.concatenate(
        [src2, jnp.zeros((G_PAD - G_CHUNKS, 1, CHUNK), jnp.int32)], axis=0
    )
    dst2 = edge_label_index[1].reshape(G_CHUNKS, 1, CHUNK)
    zeros = jnp.zeros((N_PAD, D), jnp.float32)
    partials = _edge_accumulate(h, src2, dst2, zeros)[:, :N_NODES, :]

    s = pl.pallas_call(
        _mm_sum_body,
        grid=(N_ROW_BLOCKS,),
        in_specs=[
            pl.BlockSpec((ROW_BLOCK, D), lambda i: (i, 0)),
            pl.BlockSpec((ROW_BLOCK, D), lambda i: (i, 0)),
            pl.BlockSpec((D, D), lambda i: (0, 0)),
        ],
        out_specs=pl.BlockSpec((ROW_BLOCK, D), lambda i: (i, 0)),
        out_shape=jax.ShapeDtypeStruct((N_NODES, D), jnp.float32),
    )(partials[0], partials[1], W2)
    return s
